# baseline (speedup 1.0000x reference)
import functools

import jax
import jax.numpy as jnp
from jax import lax
from jax.experimental import pallas as pl
from jax.experimental.pallas import tpu as pltpu
from jax.experimental.pallas import tpu_sc as plsc

_CHUNK = 128


@functools.lru_cache(maxsize=None)
def _make_gather(num_entities, batch, dim, nc, ns):
    nw = nc * ns
    b_per_w = batch // nw
    n_chunks = b_per_w // _CHUNK
    mesh = plsc.VectorSubcoreMesh(core_axis_name="c", subcore_axis_name="s")

    @functools.partial(
        pl.kernel,
        out_type=jax.ShapeDtypeStruct((batch, dim), jnp.float32),
        mesh=mesh,
        scratch_types=[
            pltpu.VMEM((n_chunks, _CHUNK), jnp.int32),
            pltpu.VMEM((b_per_w, dim), jnp.float32),
            pltpu.SemaphoreType.DMA,
        ],
        compiler_params=pltpu.CompilerParams(use_tc_tiling_on_sc=False),
    )
    def gather_kernel(idx_hbm, table_hbm, out_hbm, idx_v, rows_v, sem):
        wid = lax.axis_index("s") * nc + lax.axis_index("c")
        base = wid * b_per_w
        pltpu.sync_copy(idx_hbm.at[wid], idx_v)
        copies = [
            pltpu.async_copy(
                table_hbm.at[idx_v.at[j]],
                rows_v.at[pl.ds(j * _CHUNK, _CHUNK)],
                sem,
            )
            for j in range(n_chunks)
        ]
        for c in copies:
            c.wait()
        pltpu.sync_copy(rows_v, out_hbm.at[pl.ds(base, b_per_w)])

    return gather_kernel


def kernel(elements, entity_embeddings):
    (batch,) = elements.shape
    num_entities, dim = entity_embeddings.shape
    info = plsc.get_sparse_core_info()
    nc, ns = info.num_cores, info.num_subcores
    nw = nc * ns
    idx = elements.reshape(nw, batch // (nw * _CHUNK), _CHUNK)
    fn = _make_gather(num_entities, batch, dim, nc, ns)
    return fn(idx, entity_embeddings)


# speedup vs baseline: 1.7365x; 1.7365x over previous
import functools

import jax
import jax.numpy as jnp
from jax import lax
from jax.experimental import pallas as pl
from jax.experimental.pallas import tpu as pltpu
from jax.experimental.pallas import tpu_sc as plsc


@functools.lru_cache(maxsize=None)
def _make_gather(num_entities, batch, dim, nc, ns):
    nw = nc * ns
    b_per_w = batch // nw
    mesh = plsc.VectorSubcoreMesh(core_axis_name="c", subcore_axis_name="s")

    @functools.partial(
        pl.kernel,
        out_type=jax.ShapeDtypeStruct((batch, dim), jnp.float32),
        mesh=mesh,
        scratch_types=[
            pltpu.VMEM((b_per_w,), jnp.int32),
            pltpu.VMEM((b_per_w, dim), jnp.float32),
            pltpu.SemaphoreType.DMA,
        ],
    )
    def gather_kernel(idx_hbm, table_hbm, out_hbm, idx_v, rows_v, sem):
        wid = lax.axis_index("s") * nc + lax.axis_index("c")
        base = wid * b_per_w
        pltpu.sync_copy(idx_hbm.at[pl.ds(base, b_per_w)], idx_v)

        @pl.loop(0, b_per_w // 16)
        def _issue(g):
            vec = idx_v[pl.ds(g * 16, 16)]
            for l in range(16):
                pltpu.async_copy(
                    table_hbm.at[vec[l]], rows_v.at[g * 16 + l], sem
                )

        pltpu.make_async_copy(
            table_hbm.at[pl.ds(0, b_per_w)], rows_v, sem
        ).wait()
        pltpu.sync_copy(rows_v, out_hbm.at[pl.ds(base, b_per_w)])

    return gather_kernel


def kernel(elements, entity_embeddings):
    (batch,) = elements.shape
    num_entities, dim = entity_embeddings.shape
    info = plsc.get_sparse_core_info()
    fn = _make_gather(num_entities, batch, dim, info.num_cores, info.num_subcores)
    return fn(elements, entity_embeddings)


# speedup vs baseline: 1.8389x; 1.0590x over previous
import functools

import jax
import jax.numpy as jnp
from jax import lax
from jax.experimental import pallas as pl
from jax.experimental.pallas import tpu as pltpu
from jax.experimental.pallas import tpu_sc as plsc

_LANES = 16


@functools.lru_cache(maxsize=None)
def _make_gather(num_entities, batch, dim, nc, ns):
    nw = nc * ns
    b_per_w = batch // nw
    n_grp = b_per_w // _LANES
    mesh = plsc.VectorSubcoreMesh(core_axis_name="c", subcore_axis_name="s")

    @functools.partial(
        pl.kernel,
        out_type=jax.ShapeDtypeStruct((batch * dim,), jnp.float32),
        mesh=mesh,
        scratch_types=[
            pltpu.VMEM((b_per_w,), jnp.int32),
            pltpu.VMEM((dim, 128), jnp.float32),
            pltpu.VMEM((dim, 128), jnp.float32),
            pltpu.VMEM((b_per_w * dim,), jnp.float32),
            pltpu.SemaphoreType.DMA,
            pltpu.SemaphoreType.DMA,
        ],
        compiler_params=pltpu.CompilerParams(
            disable_bounds_checks=True, needs_layout_passes=False
        ),
    )
    def gather_kernel(idx_hbm, tab_hbm, out_hbm, idx_v, buf0, buf1, rows_v, s0, s1):
        wid = lax.axis_index("s") * nc + lax.axis_index("c")
        base = wid * b_per_w
        pltpu.sync_copy(idx_hbm.at[pl.ds(base, b_per_w)], idx_v)
        bufs = (buf0, buf1)
        sems = (s0, s1)
        iota = lax.iota(jnp.int32, _LANES)

        def start(v, par):
            off = pl.multiple_of((v >> 7) * 128, 128)
            pltpu.async_copy(tab_hbm.at[:, pl.ds(off, 128)], bufs[par], sems[par])

        def finish(j, v, par):
            pltpu.make_async_copy(
                tab_hbm.at[:, pl.ds(0, 128)], bufs[par], sems[par]
            ).wait()
            lane = jnp.full((_LANES,), v & 127, jnp.int32)
            for k in range(dim // _LANES):
                vals = plsc.load_gather(bufs[par], [iota + (k * _LANES), lane])
                rows_v[pl.ds(j * dim + k * _LANES, _LANES)] = vals

        vec0 = idx_v[pl.ds(0, _LANES)]
        start(vec0[0], 0)

        @pl.loop(0, n_grp, init_carry=vec0)
        def _grp(g, vec):
            nxt_off = jnp.minimum((g + 1) * _LANES, b_per_w - _LANES)
            vec_n = idx_v[pl.ds(nxt_off, _LANES)]
            for l in range(_LANES):
                j = g * _LANES + l
                if l < _LANES - 1:
                    start(vec[l + 1], (l + 1) % 2)
                else:

                    @pl.when(g < n_grp - 1)
                    def _():
                        start(vec_n[0], (l + 1) % 2)

                finish(j, vec[l], l % 2)
            return vec_n

        pltpu.sync_copy(rows_v, out_hbm.at[pl.ds(base * dim, b_per_w * dim)])

    return gather_kernel


def kernel(elements, entity_embeddings):
    (batch,) = elements.shape
    num_entities, dim = entity_embeddings.shape
    info = plsc.get_sparse_core_info()
    fn = _make_gather(num_entities, batch, dim, info.num_cores, info.num_subcores)
    flat = fn(elements, entity_embeddings.T)
    return flat.reshape(batch, dim)


# speedup vs baseline: 2.9606x; 1.6100x over previous
import functools

import jax
import jax.numpy as jnp
from jax import lax
from jax.experimental import pallas as pl
from jax.experimental.pallas import tpu as pltpu
from jax.experimental.pallas import tpu_sc as plsc

_LANES = 16


@functools.lru_cache(maxsize=None)
def _make_gather(num_entities, batch, dim, nc, ns):
    nw = nc * ns
    b_per_w = batch // nw
    n_grp = b_per_w // _LANES
    mesh = plsc.VectorSubcoreMesh(core_axis_name="c", subcore_axis_name="s")

    @functools.partial(
        pl.kernel,
        out_type=jax.ShapeDtypeStruct((batch * dim,), jnp.float32),
        mesh=mesh,
        scratch_types=(
            [pltpu.VMEM((b_per_w,), jnp.int32)]
            + [pltpu.VMEM((dim, 128), jnp.float32) for _ in range(8)]
            + [pltpu.VMEM((b_per_w * dim,), jnp.float32)]
            + [pltpu.SemaphoreType.DMA for _ in range(8)]
        ),
        compiler_params=pltpu.CompilerParams(
            disable_bounds_checks=True, needs_layout_passes=False
        ),
    )
    def gather_kernel(idx_hbm, tab_hbm, out_hbm, *refs):
        idx_v = refs[0]
        bufs = refs[1:9]
        rows_v = refs[9]
        sems = refs[10:18]
        ndeep = 8
        wid = lax.axis_index("s") * nc + lax.axis_index("c")
        base = wid * b_per_w
        pltpu.sync_copy(idx_hbm.at[pl.ds(base, b_per_w)], idx_v)
        iota = lax.iota(jnp.int32, _LANES)

        def start(v, par):
            off = pl.multiple_of((v >> 7) * 128, 128)
            pltpu.async_copy(tab_hbm.at[:, pl.ds(off, 128)], bufs[par], sems[par])

        def finish(j, v, par):
            pltpu.make_async_copy(
                tab_hbm.at[:, pl.ds(0, 128)], bufs[par], sems[par]
            ).wait()
            lane = jnp.full((_LANES,), v & 127, jnp.int32)
            for k in range(dim // _LANES):
                vals = plsc.load_gather(bufs[par], [iota + (k * _LANES), lane])
                rows_v[pl.ds(j * dim + k * _LANES, _LANES)] = vals

        vec0 = idx_v[pl.ds(0, _LANES)]
        for l in range(ndeep):
            start(vec0[l], l)

        @pl.loop(0, n_grp, init_carry=vec0)
        def _grp(g, vec):
            nxt_off = jnp.minimum((g + 1) * _LANES, b_per_w - _LANES)
            vec_n = idx_v[pl.ds(nxt_off, _LANES)]
            for l in range(_LANES):
                j = g * _LANES + l
                if l < _LANES - ndeep:
                    start(vec[l + ndeep], (l + ndeep) % ndeep)
                else:

                    @pl.when(g < n_grp - 1)
                    def _():
                        start(vec_n[l + ndeep - _LANES], (l + ndeep) % ndeep)

                finish(j, vec[l], l % ndeep)
            return vec_n

        pltpu.sync_copy(rows_v, out_hbm.at[pl.ds(base * dim, b_per_w * dim)])

    return gather_kernel


def kernel(elements, entity_embeddings):
    (batch,) = elements.shape
    num_entities, dim = entity_embeddings.shape
    info = plsc.get_sparse_core_info()
    fn = _make_gather(num_entities, batch, dim, info.num_cores, info.num_subcores)
    flat = fn(elements, entity_embeddings.T)
    return flat.reshape(batch, dim)
